# baseline (speedup 1.0000x reference)
import numpy as np
import jax
import jax.numpy as jnp
from jax.experimental import pallas as pl
from jax.experimental.pallas import tpu as pltpu

D = 48
H = 2
HD = D // H
N = 21
ND = N * D
NDP = 1024
ITERS = 3
EPS = 1e-5

_SEG = np.kron(np.eye(N, dtype=np.float32), np.ones((D, 1), np.float32))
_S2 = np.zeros((2 * NDP, 2 * N), np.float32)
_S2[:ND, :N] = _SEG
_S2[NDP:NDP + ND, N:] = _SEG
_ST2 = np.zeros((2 * N, 2 * NDP), np.float32)
_ST2[:N, :ND] = _SEG.T
_ST2[N:, NDP:NDP + ND] = _SEG.T

_G0A, _G0B, _VTA, _VTB = 0, NDP, 2 * NDP, 3 * NDP + ND
_VTB = 3 * NDP
_OC, _ISIG, _B1, _B2, _Q0 = 4224, 4352, 4480, 4608, 4736
_BIH, _BHH, _GH0 = 4864, 5120, 5376
_SCAL = 5632
_CR1_W = 5760
_MTA, _MTB, _WIH, _WHH, _P1, _P2 = 0, NDP, 2048, 2304, 2560, 2688
_M1C, _CPC = 2816, 2944
_C48_W = 3072


def _body(kv_ref, cr1_ref, c48_ref, s2_ref, st2_ref, p2v_ref,
          fused_ref, aww_ref, t_ref, u_ref):
    f32 = jnp.float32
    kv = kv_ref[...]

    zpad = jnp.zeros((kv.shape[0], NDP - ND), f32)
    t_ref[:, ND:NDP] = zpad
    t_ref[:, NDP + ND:] = zpad
    u_ref[:, ND:NDP] = zpad
    u_ref[:, NDP + ND:] = zpad

    Sm = s2_ref[0:ND, 0:N]
    ssum = jnp.dot(kv, Sm, preferred_element_type=f32)
    ssq = jnp.dot(kv * kv, Sm, preferred_element_type=f32)
    mu = ssum * (1.0 / D)
    var = ssq * (1.0 / D) - mu * mu
    rstd = jax.lax.rsqrt(var + EPS)
    rmu = rstd * mu

    cpa = c48_ref[0:1, _CPC:_CPC + D]
    cpb = c48_ref[1:2, _CPC:_CPC + D]
    oc = cr1_ref[0:1, _OC:_OC + D]
    b0a = cr1_ref[0, _SCAL]
    b0b = cr1_ref[0, _SCAL + 1]
    c1a = cr1_ref[0, _SCAL + 2]
    c1b = cr1_ref[0, _SCAL + 3]

    def softmax_n(s):
        m = jnp.max(s, axis=-1, keepdims=True)
        e = jnp.exp(s - m)
        return e / jnp.sum(e, axis=-1, keepdims=True)

    def attn_out(ga, gb, ba, bb):
        t_ref[:, 0:ND] = kv * ga
        t_ref[:, NDP:NDP + ND] = kv * gb
        A01 = jnp.dot(t_ref[...], s2_ref[...], preferred_element_type=f32)
        s0 = rstd * A01[:, 0:N] - rmu * ba
        s1 = rstd * A01[:, N:] - rmu * bb
        aw0 = softmax_n(s0)
        aw1 = softmax_n(s1)
        c0 = aw0 * rstd
        c1 = aw1 * rstd
        c01 = jnp.concatenate([c0, c1], axis=1)
        e01 = jnp.dot(c01, st2_ref[...], preferred_element_type=f32)
        k0 = jnp.sum(c0 * mu, axis=-1, keepdims=True)
        k1 = jnp.sum(c1 * mu, axis=-1, keepdims=True)
        u_ref[:, 0:ND] = kv * e01[:, 0:ND]
        u_ref[:, NDP:NDP + ND] = kv * e01[:, NDP:NDP + ND]
        out = (jnp.dot(u_ref[...], p2v_ref[...], preferred_element_type=f32)
               - k0 * cpa - k1 * cpb + oc)
        return out, aw0, aw1

    def gru(out, gh, q):
        gi = jnp.dot(out, c48_ref[:, _WIH:_WIH + 3 * D],
                     preferred_element_type=f32) \
            + cr1_ref[0:1, _BIH:_BIH + 3 * D]
        r = jax.nn.sigmoid(gi[:, :D] + gh[:, :D])
        zg = jax.nn.sigmoid(gi[:, D:2 * D] + gh[:, D:2 * D])
        n = jnp.tanh(gi[:, 2 * D:] + r * gh[:, 2 * D:])
        return (1.0 - zg) * n + zg * q

    out, aw0, aw1 = attn_out(cr1_ref[0:1, _G0A:_G0A + ND],
                             cr1_ref[0:1, _G0B:_G0B + ND], b0a, b0b)
    q = gru(out, cr1_ref[0:1, _GH0:_GH0 + 3 * D],
            cr1_ref[0:1, _Q0:_Q0 + D])

    for _ in range(ITERS - 1):
        qmu = jnp.mean(q, axis=-1, keepdims=True)
        qc = q - qmu
        qvar = jnp.mean(qc * qc, axis=-1, keepdims=True)
        zq = qc * jax.lax.rsqrt(qvar + EPS)
        gab = jnp.dot(zq, c48_ref[:, 0:2 * NDP], preferred_element_type=f32)
        ga = gab[:, 0:ND] + cr1_ref[0:1, _VTA:_VTA + ND]
        gb = gab[:, NDP:NDP + ND] + cr1_ref[0:1, _VTB:_VTB + ND]
        ba = jnp.sum(zq * c48_ref[0:1, _M1C:_M1C + D], axis=-1,
                     keepdims=True) + c1a
        bb = jnp.sum(zq * c48_ref[1:2, _M1C:_M1C + D], axis=-1,
                     keepdims=True) + c1b
        out, aw0, aw1 = attn_out(ga, gb, ba, bb)
        gh = jnp.dot(q, c48_ref[:, _WHH:_WHH + 3 * D],
                     preferred_element_type=f32) \
            + cr1_ref[0:1, _BHH:_BHH + 3 * D]
        q = gru(out, gh, q)

    aww_ref[...] = (aw0 + aw1) * 0.5

    p_s = cr1_ref[0, _SCAL + 4]
    nip_s = cr1_ref[0, _SCAL + 5]
    ratio = jnp.minimum(jnp.abs(q) * cr1_ref[0:1, _ISIG:_ISIG + D], 15.0)
    rp = jnp.exp2(p_s * jnp.log2(jnp.maximum(ratio, 1e-30)))
    f = q * jnp.exp2(nip_s * jnp.log2(1.0 + rp))

    f = jnp.maximum(
        jnp.dot(f, c48_ref[:, _P1:_P1 + D], preferred_element_type=f32)
        + cr1_ref[0:1, _B1:_B1 + D], 0.0)
    fused_ref[...] = jnp.dot(f, c48_ref[:, _P2:_P2 + D],
                             preferred_element_type=f32) \
        + cr1_ref[0:1, _B2:_B2 + D]


def kernel(slot_outputs, fusion_query, in_proj_w, in_proj_b, out_proj_w,
           out_proj_b, ln_q_g, ln_q_b, ln_kv_g, ln_kv_b, gru_w_ih,
           gru_w_hh, gru_b_ih, gru_b_hh, sigma_y_raw, p_raw,
           proj1_w, proj1_b, proj2_w, proj2_b):
    B, T, _, _ = slot_outputs.shape
    M = B * T
    f32 = jnp.float32

    wq, wk, wv = in_proj_w[:D], in_proj_w[D:2 * D], in_proj_w[2 * D:]
    bq = in_proj_b[:D]
    bv = in_proj_b[2 * D:]
    WoT = out_proj_w.T
    scale = 1.0 / np.sqrt(HD)

    def head_mats(h):
        sl = slice(h * HD, (h + 1) * HD)
        wqh, wkh, wvh = wq[sl], wk[sl], wv[sl]
        Mh = scale * (ln_q_g[:, None] * (wqh.T @ wkh)) * ln_kv_g[None, :]
        vh = scale * (((wqh @ ln_q_b + bq[sl]) @ wkh) * ln_kv_g)
        Ph = (ln_kv_g[:, None] * wvh.T) @ WoT[sl]
        return Mh, vh, Ph

    M0, v0, P0 = head_mats(0)
    M1, v1, P1 = head_mats(1)
    out_const = (out_proj_b + (ln_kv_b @ wv.T + bv) @ WoT)[None]

    fq = fusion_query
    mu0 = fq.mean()
    cq0 = fq - mu0
    zq0 = cq0 * jax.lax.rsqrt((cq0 * cq0).mean() + EPS)
    g0ah = zq0 @ M0 + v0
    g0bh = zq0 @ M1 + v1

    sigma_y = jax.nn.softplus(sigma_y_raw) + 0.01
    isig = 1.0 / sigma_y
    p = 1.5 + jax.nn.softplus(p_raw)

    cr1 = jnp.zeros((1, _CR1_W), f32)
    cr1 = cr1.at[0, _G0A:_G0A + ND].set(jnp.tile(g0ah, N))
    cr1 = cr1.at[0, _G0B:_G0B + ND].set(jnp.tile(g0bh, N))
    cr1 = cr1.at[0, _VTA:_VTA + ND].set(jnp.tile(v0, N))
    cr1 = cr1.at[0, _VTB:_VTB + ND].set(jnp.tile(v1, N))
    cr1 = cr1.at[0, _OC:_OC + D].set(out_const[0])
    cr1 = cr1.at[0, _ISIG:_ISIG + D].set(isig)
    cr1 = cr1.at[0, _B1:_B1 + D].set(proj1_b)
    cr1 = cr1.at[0, _B2:_B2 + D].set(proj2_b)
    cr1 = cr1.at[0, _Q0:_Q0 + D].set(fq)
    cr1 = cr1.at[0, _BIH:_BIH + 3 * D].set(gru_b_ih)
    cr1 = cr1.at[0, _BHH:_BHH + 3 * D].set(gru_b_hh)
    cr1 = cr1.at[0, _GH0:_GH0 + 3 * D].set(fq @ gru_w_hh.T + gru_b_hh)
    cr1 = cr1.at[0, _SCAL].set(g0ah.sum())
    cr1 = cr1.at[0, _SCAL + 1].set(g0bh.sum())
    cr1 = cr1.at[0, _SCAL + 2].set(v0.sum())
    cr1 = cr1.at[0, _SCAL + 3].set(v1.sum())
    cr1 = cr1.at[0, _SCAL + 4].set(p[0])
    cr1 = cr1.at[0, _SCAL + 5].set(-1.0 / p[0])

    c48 = jnp.zeros((D, _C48_W), f32)
    c48 = c48.at[:, _MTA:_MTA + ND].set(jnp.tile(M0, (1, N)))
    c48 = c48.at[:, _MTB:_MTB + ND].set(jnp.tile(M1, (1, N)))
    c48 = c48.at[:, _WIH:_WIH + 3 * D].set(gru_w_ih.T)
    c48 = c48.at[:, _WHH:_WHH + 3 * D].set(gru_w_hh.T)
    c48 = c48.at[:, _P1:_P1 + D].set(proj1_w.T)
    c48 = c48.at[:, _P2:_P2 + D].set(proj2_w.T)
    c48 = c48.at[0, _M1C:_M1C + D].set(M0.sum(axis=1))
    c48 = c48.at[1, _M1C:_M1C + D].set(M1.sum(axis=1))
    c48 = c48.at[0, _CPC:_CPC + D].set(P0.sum(axis=0))
    c48 = c48.at[1, _CPC:_CPC + D].set(P1.sum(axis=0))

    p2v = jnp.zeros((2 * NDP, D), f32)
    p2v = p2v.at[0:ND].set(jnp.tile(P0, (N, 1)))
    p2v = p2v.at[NDP:NDP + ND].set(jnp.tile(P1, (N, 1)))

    kv2 = slot_outputs.reshape(M, ND)
    R = 1024
    while M % R:
        R //= 2
    grid = (M // R,)

    def const(shape):
        return pl.BlockSpec(shape, lambda i: (0, 0))

    fused, aww = pl.pallas_call(
        _body,
        grid=grid,
        in_specs=[
            pl.BlockSpec((R, ND), lambda i: (i, 0)),
            const((1, _CR1_W)),
            const((D, _C48_W)),
            const((2 * NDP, 2 * N)),
            const((2 * N, 2 * NDP)),
            const((2 * NDP, D)),
        ],
        out_specs=[
            pl.BlockSpec((R, D), lambda i: (i, 0)),
            pl.BlockSpec((R, N), lambda i: (i, 0)),
        ],
        out_shape=[
            jax.ShapeDtypeStruct((M, D), f32),
            jax.ShapeDtypeStruct((M, N), f32),
        ],
        scratch_shapes=[
            pltpu.VMEM((R, 2 * NDP), f32),
            pltpu.VMEM((R, 2 * NDP), f32),
        ],
        compiler_params=pltpu.CompilerParams(
            dimension_semantics=("arbitrary",),
            vmem_limit_bytes=48 * 1024 * 1024),
    )(kv2, cr1, c48, jnp.asarray(_S2), jnp.asarray(_ST2), p2v)

    return fused.reshape(B, T, D), aww.reshape(B, T, N)


# speedup vs baseline: 1.5194x; 1.5194x over previous
import numpy as np
import jax
import jax.numpy as jnp
from jax.experimental import pallas as pl
from jax.experimental.pallas import tpu as pltpu

D = 48
H = 2
HD = D // H
N = 21
ND = N * D
ITERS = 3
EPS = 1e-5

_SEG = np.kron(np.eye(N, dtype=np.float32), np.ones((D, 1), np.float32))


def _body(kv_ref, g0a_ref, g0b_ref, mta_ref, mtb_ref, vta_ref, vtb_ref,
          pva_ref, pvb_ref, s_ref, st_ref, oc_ref, wih_ref, bih_ref,
          whh_ref, bhh_ref, gh0_ref, q0_ref, isig_ref, p_ref, nip_ref,
          p1_ref, b1_ref, p2_ref, b2_ref, fused_ref, aww_ref):
    f32 = jnp.float32
    kv = kv_ref[...]
    S = s_ref[...]
    ST = st_ref[...]

    ssum = jnp.dot(kv, S, preferred_element_type=f32)
    ssq = jnp.dot(kv * kv, S, preferred_element_type=f32)
    mu = ssum * (1.0 / D)
    var = ssq * (1.0 / D) - mu * mu
    rstd = jax.lax.rsqrt(var + EPS)
    z = (kv - jnp.dot(mu, ST, preferred_element_type=f32)) \
        * jnp.dot(rstd, ST, preferred_element_type=f32)

    def softmax_n(s):
        m = jnp.max(s, axis=-1, keepdims=True)
        e = jnp.exp(s - m)
        return e / jnp.sum(e, axis=-1, keepdims=True)

    def attn_out(ga, gb):
        s0 = jnp.dot(z * ga, S, preferred_element_type=f32)
        s1 = jnp.dot(z * gb, S, preferred_element_type=f32)
        aw0 = softmax_n(s0)
        aw1 = softmax_n(s1)
        e0 = jnp.dot(aw0, ST, preferred_element_type=f32)
        e1 = jnp.dot(aw1, ST, preferred_element_type=f32)
        out = (jnp.dot(z * e0, pva_ref[...], preferred_element_type=f32)
               + jnp.dot(z * e1, pvb_ref[...], preferred_element_type=f32)
               + oc_ref[...])
        return out, aw0, aw1

    def gru(out, gh, q):
        gi = jnp.dot(out, wih_ref[...], preferred_element_type=f32) \
            + bih_ref[...]
        r = jax.nn.sigmoid(gi[:, :D] + gh[:, :D])
        zg = jax.nn.sigmoid(gi[:, D:2 * D] + gh[:, D:2 * D])
        n = jnp.tanh(gi[:, 2 * D:] + r * gh[:, 2 * D:])
        return (1.0 - zg) * n + zg * q

    out, aw0, aw1 = attn_out(g0a_ref[...], g0b_ref[...])
    q = gru(out, gh0_ref[...], q0_ref[...])

    for _ in range(ITERS - 1):
        qmu = jnp.mean(q, axis=-1, keepdims=True)
        qc = q - qmu
        qvar = jnp.mean(qc * qc, axis=-1, keepdims=True)
        zq = qc * jax.lax.rsqrt(qvar + EPS)
        ga = jnp.dot(zq, mta_ref[...], preferred_element_type=f32) \
            + vta_ref[...]
        gb = jnp.dot(zq, mtb_ref[...], preferred_element_type=f32) \
            + vtb_ref[...]
        out, aw0, aw1 = attn_out(ga, gb)
        gh = jnp.dot(q, whh_ref[...], preferred_element_type=f32) \
            + bhh_ref[...]
        q = gru(out, gh, q)

    aww_ref[...] = (aw0 + aw1) * 0.5

    ratio = jnp.minimum(jnp.abs(q) * isig_ref[...], 15.0)
    rp = jnp.exp2(p_ref[...] * jnp.log2(jnp.maximum(ratio, 1e-30)))
    f = q * jnp.exp2(nip_ref[...] * jnp.log2(1.0 + rp))

    f = jnp.maximum(
        jnp.dot(f, p1_ref[...], preferred_element_type=f32) + b1_ref[...],
        0.0)
    fused_ref[...] = jnp.dot(f, p2_ref[...], preferred_element_type=f32) \
        + b2_ref[...]


def kernel(slot_outputs, fusion_query, in_proj_w, in_proj_b, out_proj_w,
           out_proj_b, ln_q_g, ln_q_b, ln_kv_g, ln_kv_b, gru_w_ih,
           gru_w_hh, gru_b_ih, gru_b_hh, sigma_y_raw, p_raw,
           proj1_w, proj1_b, proj2_w, proj2_b):
    B, T, _, _ = slot_outputs.shape
    M = B * T
    f32 = jnp.float32

    wq, wk, wv = in_proj_w[:D], in_proj_w[D:2 * D], in_proj_w[2 * D:]
    bq = in_proj_b[:D]
    bv = in_proj_b[2 * D:]
    WoT = out_proj_w.T
    scale = 1.0 / np.sqrt(HD)

    def head_mats(h):
        sl = slice(h * HD, (h + 1) * HD)
        wqh, wkh, wvh = wq[sl], wk[sl], wv[sl]
        Mh = scale * (ln_q_g[:, None] * (wqh.T @ wkh)) * ln_kv_g[None, :]
        vh = scale * (((wqh @ ln_q_b + bq[sl]) @ wkh) * ln_kv_g)
        Ph = (ln_kv_g[:, None] * wvh.T) @ WoT[sl]
        return Mh, vh, Ph

    M0, v0, P0 = head_mats(0)
    M1, v1, P1 = head_mats(1)
    Mta = jnp.tile(M0, (1, N))
    Mtb = jnp.tile(M1, (1, N))
    vta = jnp.tile(v0, N)[None]
    vtb = jnp.tile(v1, N)[None]
    Pva = jnp.tile(P0, (N, 1))
    Pvb = jnp.tile(P1, (N, 1))
    out_const = (out_proj_b + (ln_kv_b @ wv.T + bv) @ WoT)[None]

    fq = fusion_query
    mu0 = fq.mean()
    c0 = fq - mu0
    zq0 = c0 * jax.lax.rsqrt((c0 * c0).mean() + EPS)
    g0a = (zq0 @ Mta + vta)
    g0b = (zq0 @ Mtb + vtb)
    gh0 = (fq @ gru_w_hh.T + gru_b_hh)[None]
    q0 = fq[None]

    sigma_y = jax.nn.softplus(sigma_y_raw) + 0.01
    isig = (1.0 / sigma_y)[None]
    p = 1.5 + jax.nn.softplus(p_raw)
    p_arr = p[:, None]
    nip = (-1.0 / p)[:, None]

    kv2 = slot_outputs.reshape(M, ND)
    R = 1024
    while M % R:
        R //= 2
    grid = (M // R,)

    def const(shape):
        return pl.BlockSpec(shape, lambda i: (0, 0))

    fused, aww = pl.pallas_call(
        _body,
        grid=grid,
        in_specs=[
            pl.BlockSpec((R, ND), lambda i: (i, 0)),
            const((1, ND)), const((1, ND)),
            const((D, ND)), const((D, ND)),
            const((1, ND)), const((1, ND)),
            const((ND, D)), const((ND, D)),
            const((ND, N)), const((N, ND)),
            const((1, D)),
            const((D, 3 * D)), const((1, 3 * D)),
            const((D, 3 * D)), const((1, 3 * D)),
            const((1, 3 * D)), const((1, D)),
            const((1, D)), const((1, 1)), const((1, 1)),
            const((D, D)), const((1, D)), const((D, D)), const((1, D)),
        ],
        out_specs=[
            pl.BlockSpec((R, D), lambda i: (i, 0)),
            pl.BlockSpec((R, N), lambda i: (i, 0)),
        ],
        out_shape=[
            jax.ShapeDtypeStruct((M, D), f32),
            jax.ShapeDtypeStruct((M, N), f32),
        ],
        compiler_params=pltpu.CompilerParams(
            dimension_semantics=("arbitrary",),
            vmem_limit_bytes=48 * 1024 * 1024),
    )(kv2, g0a, g0b, Mta, Mtb, vta, vtb, Pva, Pvb,
      jnp.asarray(_SEG), jnp.asarray(_SEG.T), out_const,
      gru_w_ih.T, gru_b_ih[None], gru_w_hh.T, gru_b_hh[None], gh0, q0,
      isig, p_arr, nip,
      proj1_w.T, proj1_b[None], proj2_w.T, proj2_b[None])

    return fused.reshape(B, T, D), aww.reshape(B, T, N)
